# baseline (speedup 1.0000x reference)
import jax
import jax.numpy as jnp
from jax.experimental import pallas as pl
from jax.experimental.pallas import tpu as pltpu

POP = 4096
KEY_DIM = 1024
HIDDEN = 64
BLOCK = 256


def _brkga_fitness_kernel(x_ref, w_ref, out_ref):
    h = jnp.dot(x_ref[...], w_ref[...], preferred_element_type=jnp.float32)
    out_ref[...] = jnp.sum(jnp.maximum(h, 0.0), axis=1)


def kernel(keys_pop, W):
    grid = (POP // BLOCK,)
    return pl.pallas_call(
        _brkga_fitness_kernel,
        grid=grid,
        in_specs=[
            pl.BlockSpec((BLOCK, KEY_DIM), lambda i: (i, 0)),
            pl.BlockSpec((KEY_DIM, HIDDEN), lambda i: (0, 0)),
        ],
        out_specs=pl.BlockSpec((BLOCK,), lambda i: (i,)),
        out_shape=jax.ShapeDtypeStruct((POP,), jnp.float32),
        compiler_params=pltpu.CompilerParams(
            dimension_semantics=("parallel",),
        ),
    )(keys_pop, W)


# speedup vs baseline: 1.3185x; 1.3185x over previous
import jax
import jax.numpy as jnp
from jax.experimental import pallas as pl
from jax.experimental.pallas import tpu as pltpu

POP = 4096
KEY_DIM = 1024
HIDDEN = 64
BLOCK = 1024


def _brkga_fitness_kernel(x_ref, w_ref, out_ref):
    h = jnp.dot(x_ref[...], w_ref[...], preferred_element_type=jnp.float32)
    out_ref[...] = jnp.sum(jnp.maximum(h, 0.0), axis=1)


def kernel(keys_pop, W):
    grid = (POP // BLOCK,)
    return pl.pallas_call(
        _brkga_fitness_kernel,
        grid=grid,
        in_specs=[
            pl.BlockSpec((BLOCK, KEY_DIM), lambda i: (i, 0)),
            pl.BlockSpec((KEY_DIM, HIDDEN), lambda i: (0, 0)),
        ],
        out_specs=pl.BlockSpec((BLOCK,), lambda i: (i,)),
        out_shape=jax.ShapeDtypeStruct((POP,), jnp.float32),
        compiler_params=pltpu.CompilerParams(
            dimension_semantics=("parallel",),
        ),
    )(keys_pop, W)
